# Initial kernel scaffold; baseline (speedup 1.0000x reference)
import jax
import jax.numpy as jnp
from jax.experimental import pallas as pl


def kernel(x_Patient, x_Admission, edges, params):
    raise NotImplementedError("write your pallas kernel here")


# speedup vs baseline: 4.9610x; 4.9610x over previous
import functools
import math

import jax
import jax.numpy as jnp
from jax import lax
from jax.experimental import pallas as pl
from jax.experimental.pallas import tpu as pltpu
from jax.experimental.pallas import tpu_sc as plsc

HID = 64
NC, NS = 2, 16
CHUNK = 1024
NSUB = 8
SUB = 128

_NTYPES = ["Patient", "Admission", "Diagnosis", "Medication", "Procedure", "LabTest"]
_RELS = [
    ("Patient", "Admission"),
    ("Admission", "Patient"),
    ("Admission", "Diagnosis"),
    ("Diagnosis", "Admission"),
    ("Admission", "Medication"),
    ("Medication", "Admission"),
    ("Admission", "Procedure"),
    ("Procedure", "Admission"),
    ("Admission", "LabTest"),
    ("LabTest", "Admission"),
]
_PADN = {"Patient": 10240, "Admission": 50176, "Diagnosis": 2048,
         "Medication": 1024, "Procedure": 2048, "LabTest": 1024}
HADM = _PADN["Admission"] // 2
TRASH_F = 256
TRASH_R = 8

_INV_BN = 1.0 / math.sqrt(1.0 + 1e-5)

_FWD = [(s, d) for (s, d) in _RELS if d == "Admission"]
_REV = [(s, d) for (s, d) in _RELS if s == "Admission"]


def _rk(s, d):
    return s + "__" + d


def _cdiv(a, b):
    return -(-a // b)


def _fill_zeros(ref, rows, width):
    z = jnp.zeros((16,), jnp.float32)
    for i in range(rows):
        for jj in range(width // 16):
            ref[i, pl.ds(jj * 16, 16)] = z


def _zero_spmem(acc, zbuf, zrows, base, share):
    off = 0
    while off < share:
        sz = min(zrows, share - off)
        src = zbuf if sz == zrows else zbuf.at[pl.ds(0, sz)]
        pltpu.sync_copy(src, acc.at[pl.ds(base + off, sz)])
        off += sz


def _mesh():
    return plsc.VectorSubcoreMesh(core_axis_name="c", subcore_axis_name="s",
                                  num_cores=NC, num_subcores=NS)


_SC_PARAMS = pltpu.CompilerParams(use_tc_tiling_on_sc=False)


@functools.lru_cache(maxsize=None)
def _make_fwd_seg(n_sub):
    nsub = 2
    n_chunks = n_sub // nsub
    ki = _cdiv(n_chunks, NS)
    share = HADM // NS

    @functools.partial(
        pl.kernel, mesh=_mesh(), compiler_params=_SC_PARAMS,
        out_type=jax.ShapeDtypeStruct((2 * HADM, HID), jnp.float32),
        scratch_types=[
            pltpu.VMEM((nsub, SUB), jnp.int32),
            pltpu.VMEM((nsub, SUB), jnp.int32),
            pltpu.VMEM((nsub, SUB, HID), jnp.float32),
            pltpu.VMEM((32, HID), jnp.float32),
            pltpu.VMEM_SHARED((HADM + TRASH_F, HID), jnp.float32),
            pltpu.SemaphoreType.DMA,
        ],
    )
    def k(tab, srcm, dstm, out, src_v, dst_v, rows_v, zbuf, acc, sem):
        cid = lax.axis_index("c")
        sid = lax.axis_index("s")
        _fill_zeros(zbuf, 32, HID)
        _zero_spmem(acc, zbuf, 32, sid * share, share)
        plsc.subcore_barrier()

        def chunk(kk, carry):
            c = sid + kk * NS

            @pl.when(c < n_chunks)
            def _():
                pltpu.sync_copy(srcm.at[pl.ds(c * nsub, nsub)], src_v)
                pltpu.sync_copy(dstm.at[pl.ds(c * nsub, nsub)], dst_v)
                for j in range(nsub):
                    def adj(t, cy, j=j):
                        v = dst_v[j, pl.ds(t * 16, 16)]
                        lv = v - cid * HADM
                        ok = (lv >= 0) & (lv < HADM)
                        dst_v[j, pl.ds(t * 16, 16)] = jnp.where(
                            ok, lv, HADM + (lv & (TRASH_F - 1)))
                        return cy
                    lax.fori_loop(0, SUB // 16, adj, 0)
                descs = [pltpu.async_copy(tab.at[src_v.at[j]], rows_v.at[j], sem)
                         for j in range(nsub)]
                for d_ in descs:
                    d_.wait()
                for j in range(nsub):
                    pltpu.sync_copy(rows_v.at[j], acc.at[dst_v.at[j]], add=True)
            return carry

        lax.fori_loop(0, ki, chunk, 0)
        plsc.subcore_barrier()
        pltpu.sync_copy(acc.at[pl.ds(sid * share, share)],
                        out.at[pl.ds(cid * HADM + sid * share, share)])

    return k


@functools.lru_cache(maxsize=None)
def _make_rev_seg(n_sub_tuple, n_acc_tuple):
    nsub = 4
    nrel = len(n_sub_tuple)
    w = NC * NS
    scratch = [
        pltpu.VMEM((nsub, SUB), jnp.int32),
        pltpu.VMEM((nsub, SUB), jnp.int32),
        pltpu.VMEM((nsub, SUB, HID), jnp.float32),
        pltpu.VMEM((32, HID), jnp.float32),
    ]
    for na in n_acc_tuple:
        scratch.append(pltpu.VMEM_SHARED((na + TRASH_R, HID), jnp.float32))
    scratch.append(pltpu.SemaphoreType.DMA)

    @functools.partial(
        pl.kernel, mesh=_mesh(), compiler_params=_SC_PARAMS,
        out_type=[jax.ShapeDtypeStruct((2 * na, HID), jnp.float32)
                  for na in n_acc_tuple],
        scratch_types=scratch,
    )
    def k(*refs):
        tab = refs[0]
        srcs = refs[1:1 + 2 * nrel:2]
        dsts = refs[2:2 + 2 * nrel:2]
        outs = refs[1 + 2 * nrel:1 + 3 * nrel]
        src_v, dst_v, rows_v, zbuf = refs[1 + 3 * nrel:5 + 3 * nrel]
        accs = refs[5 + 3 * nrel:5 + 4 * nrel]

        cid = lax.axis_index("c")
        sid = lax.axis_index("s")
        wid = sid * NC + cid
        _fill_zeros(zbuf, 32, HID)
        for r in range(nrel):
            share = n_acc_tuple[r] // NS
            _zero_spmem(accs[r], zbuf, 32, sid * share, share)
        plsc.subcore_barrier()

        for r in range(nrel):
            nck = n_sub_tuple[r] // nsub
            ki = _cdiv(nck, w)

            def chunk(kk, carry, r=r, nck=nck):
                c = wid + kk * w

                @pl.when(c < nck)
                def _():
                    pltpu.sync_copy(srcs[r].at[pl.ds(c * nsub, nsub)], src_v)
                    pltpu.sync_copy(dsts[r].at[pl.ds(c * nsub, nsub)], dst_v)
                    descs = [pltpu.async_copy(tab.at[src_v.at[j]], rows_v.at[j],
                                              refs[-1])
                             for j in range(nsub)]
                    for d_ in descs:
                        d_.wait()
                    for j in range(nsub):
                        pltpu.sync_copy(rows_v.at[j], accs[r].at[dst_v.at[j]],
                                        add=True)
                return carry

            lax.fori_loop(0, ki, chunk, 0)
        plsc.subcore_barrier()
        for r in range(nrel):
            na = n_acc_tuple[r]
            share = na // NS
            pltpu.sync_copy(accs[r].at[pl.ds(sid * share, share)],
                            outs[r].at[pl.ds(cid * na + sid * share, share)])

    return k


@functools.lru_cache(maxsize=None)
def _make_degree(meta):
    cw = 16
    scratch = [
        pltpu.VMEM((NSUB, SUB), jnp.int32),
        pltpu.VMEM((SUB, cw), jnp.float32),
        pltpu.VMEM((256, cw), jnp.float32),
        pltpu.VMEM_SHARED((HADM + TRASH_F, cw), jnp.float32),
    ]
    out_type = [jax.ShapeDtypeStruct((2 * n_eff, cw), jnp.float32)
                for (_, n_eff, _) in meta]

    @functools.partial(pl.kernel, mesh=_mesh(), out_type=out_type,
                       compiler_params=_SC_PARAMS, scratch_types=scratch)
    def k(*refs):
        nrel = len(meta)
        dsts = refs[:nrel]
        outs = refs[nrel:2 * nrel]
        dst_v, ones_v, zb, cnt = refs[2 * nrel:2 * nrel + 4]
        cid = lax.axis_index("c")
        sid = lax.axis_index("s")
        wid = sid * NC + cid
        one = jnp.ones((16,), jnp.float32)
        for i in range(SUB):
            ones_v[i, pl.ds(0, 16)] = one
        _fill_zeros(zb, 256, cw)

        for r, (nsb, n_eff, is_fwd) in enumerate(meta):
            nck = nsb // NSUB
            share = n_eff // NS
            _zero_spmem(cnt, zb, 256, sid * share, share)
            plsc.subcore_barrier()
            w = NS if is_fwd else NC * NS
            wloc = sid if is_fwd else wid
            ki = _cdiv(nck, w)

            def chunk(kk, carry, r=r, nck=nck, is_fwd=is_fwd, w=w, wloc=wloc):
                c = wloc + kk * w

                @pl.when(c < nck)
                def _():
                    pltpu.sync_copy(dsts[r].at[pl.ds(c * NSUB, NSUB)], dst_v)
                    if is_fwd:
                        for j in range(NSUB):
                            def adj(t, cy, j=j):
                                v = dst_v[j, pl.ds(t * 16, 16)]
                                lv = v - cid * HADM
                                ok = (lv >= 0) & (lv < HADM)
                                dst_v[j, pl.ds(t * 16, 16)] = jnp.where(
                                    ok, lv, HADM + (lv & (TRASH_F - 1)))
                                return cy
                            lax.fori_loop(0, SUB // 16, adj, 0)
                    for j in range(NSUB):
                        pltpu.sync_copy(ones_v, cnt.at[dst_v.at[j]], add=True)
                return carry

            lax.fori_loop(0, ki, chunk, 0)
            plsc.subcore_barrier()
            pltpu.sync_copy(cnt.at[pl.ds(sid * share, share)],
                            outs[r].at[pl.ds(cid * n_eff + sid * share, share)])
            plsc.subcore_barrier()

    return k


_BN = 512


@functools.lru_cache(maxsize=None)
def _make_transform(n_pad, kdim):
    def body(x_ref, w_ref, b_ref, g_ref, b2_ref, o_ref):
        o = jnp.dot(x_ref[...], w_ref[...], preferred_element_type=jnp.float32)
        o = o + b_ref[...]
        o_ref[...] = o * (g_ref[...] * _INV_BN) + b2_ref[...]

    return pl.pallas_call(
        body,
        grid=(n_pad // _BN,),
        in_specs=[
            pl.BlockSpec((_BN, kdim), lambda i: (i, 0)),
            pl.BlockSpec((kdim, HID), lambda i: (0, 0)),
            pl.BlockSpec((1, HID), lambda i: (0, 0)),
            pl.BlockSpec((1, HID), lambda i: (0, 0)),
            pl.BlockSpec((1, HID), lambda i: (0, 0)),
        ],
        out_specs=pl.BlockSpec((_BN, HID), lambda i: (i, 0)),
        out_shape=jax.ShapeDtypeStruct((n_pad, HID), jnp.float32),
    )


@functools.lru_cache(maxsize=None)
def _make_rc_fwd(nrel):
    def body(*refs):
        for r in range(nrel):
            refs[nrel + r][...] = 1.0 / jnp.maximum(refs[r][...], 1.0)

    n = 2 * HADM
    return pl.pallas_call(
        body,
        grid=(n // _BN,),
        in_specs=[pl.BlockSpec((_BN, 16), lambda i: (i, 0))] * nrel,
        out_specs=[pl.BlockSpec((_BN, 16), lambda i: (i, 0))] * nrel,
        out_shape=[jax.ShapeDtypeStruct((n, 16), jnp.float32)] * nrel,
    )


@functools.lru_cache(maxsize=None)
def _make_rc_rev(n_acc_tuple):
    def body(*refs):
        nrel = len(n_acc_tuple)
        for r, na in enumerate(n_acc_tuple):
            c = refs[r][...]
            refs[nrel + r][...] = 1.0 / jnp.maximum(c[0] + c[1], 1.0)

    return pl.pallas_call(
        body,
        in_specs=[pl.BlockSpec((2, na, 16), lambda: (0, 0, 0))
                  for na in n_acc_tuple],
        out_specs=[pl.BlockSpec((na, 16), lambda: (0, 0))
                   for na in n_acc_tuple],
        out_shape=[jax.ShapeDtypeStruct((na, 16), jnp.float32)
                   for na in n_acc_tuple],
    )


@functools.lru_cache(maxsize=None)
def _make_combine_adm(nrel, final):
    n = 2 * HADM
    odim = 2 if final else HID

    def body(*refs):
        s_refs = refs[:nrel]
        rc_refs = refs[nrel:2 * nrel]
        x_ref = refs[2 * nrel]
        wl = refs[2 * nrel + 1][...]
        bl = refs[2 * nrel + 2][...]
        wr = refs[2 * nrel + 3][...]
        ep1 = refs[2 * nrel + 4]
        ep2 = refs[2 * nrel + 5]
        o_ref = refs[2 * nrel + 6]
        x = x_ref[...]
        best = None
        for r in range(nrel):
            a = s_refs[r][...] * rc_refs[r][...][:, 0:1]
            o = (jnp.dot(a, wl[r], preferred_element_type=jnp.float32)
                 + jnp.dot(x, wr[r], preferred_element_type=jnp.float32)
                 + bl[r][None, :])
            best = o if best is None else jnp.maximum(best, o)
        if final:
            h = jnp.maximum(best, 0.0)
            o_ref[...] = (jnp.dot(h, ep1[...], preferred_element_type=jnp.float32)
                          + ep2[...])
        else:
            h = best * (ep1[...] * _INV_BN) + ep2[...]
            o_ref[...] = jnp.maximum(h, 0.0)

    ep_specs = ([pl.BlockSpec((HID, 2), lambda i: (0, 0)),
                 pl.BlockSpec((1, 2), lambda i: (0, 0))] if final else
                [pl.BlockSpec((1, HID), lambda i: (0, 0)),
                 pl.BlockSpec((1, HID), lambda i: (0, 0))])
    return pl.pallas_call(
        body,
        grid=(n // _BN,),
        in_specs=(
            [pl.BlockSpec((_BN, HID), lambda i: (i, 0))] * nrel
            + [pl.BlockSpec((_BN, 16), lambda i: (i, 0))] * nrel
            + [pl.BlockSpec((_BN, HID), lambda i: (i, 0)),
               pl.BlockSpec((nrel, HID, HID), lambda i: (0, 0, 0)),
               pl.BlockSpec((nrel, HID), lambda i: (0, 0)),
               pl.BlockSpec((nrel, HID, HID), lambda i: (0, 0, 0))]
            + ep_specs),
        out_specs=pl.BlockSpec((_BN, odim), lambda i: (i, 0)),
        out_shape=jax.ShapeDtypeStruct((n, odim), jnp.float32),
    )


@functools.lru_cache(maxsize=None)
def _make_combine_small(n_acc, final):
    odim = 2 if final else HID

    def body(s_ref, rc_ref, x_ref, wl_ref, bl_ref, wr_ref, ep1, ep2, o_ref):
        s = s_ref[...]
        a = (s[0] + s[1]) * rc_ref[...][:, 0:1]
        o = (jnp.dot(a, wl_ref[...], preferred_element_type=jnp.float32)
             + jnp.dot(x_ref[...], wr_ref[...], preferred_element_type=jnp.float32)
             + bl_ref[...])
        if final:
            h = jnp.maximum(o, 0.0)
            o_ref[...] = (jnp.dot(h, ep1[...], preferred_element_type=jnp.float32)
                          + ep2[...])
        else:
            h = o * (ep1[...] * _INV_BN) + ep2[...]
            o_ref[...] = jnp.maximum(h, 0.0)

    ep_specs = ([pl.BlockSpec((HID, 2), lambda i: (0, 0)),
                 pl.BlockSpec((1, 2), lambda i: (0, 0))] if final else
                [pl.BlockSpec((1, HID), lambda i: (0, 0)),
                 pl.BlockSpec((1, HID), lambda i: (0, 0))])
    return pl.pallas_call(
        body,
        grid=(n_acc // _BN,),
        in_specs=(
            [pl.BlockSpec((2, _BN, HID), lambda i: (0, i, 0)),
             pl.BlockSpec((_BN, 16), lambda i: (i, 0)),
             pl.BlockSpec((_BN, HID), lambda i: (i, 0)),
             pl.BlockSpec((HID, HID), lambda i: (0, 0)),
             pl.BlockSpec((1, HID), lambda i: (0, 0)),
             pl.BlockSpec((HID, HID), lambda i: (0, 0))]
            + ep_specs),
        out_specs=pl.BlockSpec((_BN, odim), lambda i: (i, 0)),
        out_shape=jax.ShapeDtypeStruct((n_acc, odim), jnp.float32),
    )


def _pad_rows(x, n_pad):
    n = x.shape[0]
    if n == n_pad:
        return x
    return jnp.concatenate(
        [x, jnp.zeros((n_pad - n,) + x.shape[1:], x.dtype)], axis=0)


def _prep_edges(ei, trash_base, spread):
    e = ei.shape[1]
    c = _cdiv(e, CHUNK)
    ep = c * CHUNK
    npad = ep - e
    src = jnp.concatenate([ei[0], jnp.zeros((npad,), jnp.int32)])
    padv = trash_base + (jnp.arange(npad, dtype=jnp.int32) % spread)
    dst = jnp.concatenate([ei[1], padv])
    return src.reshape(c * NSUB, SUB), dst.reshape(c * NSUB, SUB), c


def kernel(x_Patient, x_Admission, edges, params):
    p = params
    emats = {}
    nchunks = {}
    for (s, d) in _RELS:
        k = _rk(s, d)
        if d == "Admission":
            tb, sp = 2 * HADM, TRASH_F
        else:
            tb, sp = _PADN[d], TRASH_R
        sm, dm, c = _prep_edges(edges[k], tb, sp)
        emats[k] = (sm, dm)
        nchunks[k] = c * NSUB

    deg_meta = tuple(
        (nchunks[_rk(s, d)], HADM if d == "Admission" else _PADN[d],
         d == "Admission")
        for (s, d) in _RELS)
    deg = _make_degree(deg_meta)(*[emats[_rk(s, d)][1] for (s, d) in _RELS])
    cnt = {_rk(s, d): deg[i] for i, (s, d) in enumerate(_RELS)}

    fwd_keys = [_rk(s, d) for (s, d) in _FWD]
    rev_keys = [_rk(s, d) for (s, d) in _REV]
    rc_f = _make_rc_fwd(len(fwd_keys))(*[cnt[k] for k in fwd_keys])
    rev_nacc = tuple(_PADN[d] for (_, d) in _REV)
    rc_r = _make_rc_rev(rev_nacc)(
        *[cnt[k].reshape(2, _PADN[d], 16) for k, (_, d) in zip(rev_keys, _REV)])
    rc = dict(zip(fwd_keys, rc_f))
    rc.update(zip(rev_keys, rc_r))

    tabs = {
        "Patient": _make_transform(_PADN["Patient"], 32)(
            _pad_rows(x_Patient, _PADN["Patient"]),
            p["pat_lin"]["W"], p["pat_lin"]["b"].reshape(1, HID),
            p["pat_bn"]["g"].reshape(1, HID), p["pat_bn"]["b"].reshape(1, HID)),
        "Admission": _make_transform(_PADN["Admission"], 48)(
            _pad_rows(x_Admission, _PADN["Admission"]),
            p["adm_lin"]["W"], p["adm_lin"]["b"].reshape(1, HID),
            p["adm_bn"]["g"].reshape(1, HID), p["adm_bn"]["b"].reshape(1, HID)),
    }
    for nt in ["Diagnosis", "Medication", "Procedure", "LabTest"]:
        tabs[nt] = _pad_rows(p["emb"][nt], _PADN[nt])

    rev_chunks = tuple(nchunks[k] for k in rev_keys)
    out_heads = None
    for layer in ["1", "2", "3"]:
        final = layer == "3"
        pconv = p["conv"][layer]
        ssum_fwd = {}
        for (s, d) in _FWD:
            k = _rk(s, d)
            sm, dm = emats[k]
            ssum_fwd[k] = _make_fwd_seg(nchunks[k])(tabs[s], sm, dm)
        rev_args = [tabs["Admission"]]
        for k in rev_keys:
            rev_args.extend(emats[k])
        rev_outs = _make_rev_seg(rev_chunks, rev_nacc)(*rev_args)
        ssum_rev = dict(zip(rev_keys, rev_outs))

        new_tabs = {}
        wl = jnp.stack([pconv[k]["Wl"] for k in fwd_keys])
        bl = jnp.stack([pconv[k]["bl"] for k in fwd_keys])
        wr = jnp.stack([pconv[k]["Wr"] for k in fwd_keys])
        if final:
            ep1 = p["lin"]["Admission"]["W"]
            ep2 = p["lin"]["Admission"]["b"].reshape(1, 2)
        else:
            ep1 = p["bn"][layer]["Admission"]["g"].reshape(1, HID)
            ep2 = p["bn"][layer]["Admission"]["b"].reshape(1, HID)
        new_tabs["Admission"] = _make_combine_adm(len(fwd_keys), final)(
            *[ssum_fwd[k] for k in fwd_keys],
            *[rc[k] for k in fwd_keys],
            tabs["Admission"], wl, bl, wr, ep1, ep2)
        for (s, d) in _REV:
            k = _rk(s, d)
            na = _PADN[d]
            if final:
                e1 = p["lin"][d]["W"]
                e2 = p["lin"][d]["b"].reshape(1, 2)
            else:
                e1 = p["bn"][layer][d]["g"].reshape(1, HID)
                e2 = p["bn"][layer][d]["b"].reshape(1, HID)
            new_tabs[d] = _make_combine_small(na, final)(
                ssum_rev[k].reshape(2, na, HID), rc[k], tabs[d],
                pconv[k]["Wl"], pconv[k]["bl"].reshape(1, HID),
                pconv[k]["Wr"], e1, e2)
        if final:
            out_heads = new_tabs
        else:
            tabs = new_tabs

    nreal = {"Patient": x_Patient.shape[0], "Admission": x_Admission.shape[0]}
    for nt in ["Diagnosis", "Medication", "Procedure", "LabTest"]:
        nreal[nt] = p["emb"][nt].shape[0]
    return tuple(out_heads[nt][:nreal[nt]] for nt in _NTYPES)


# speedup vs baseline: 6.3319x; 1.2763x over previous
import functools
import math

import jax
import jax.numpy as jnp
from jax import lax
from jax.experimental import pallas as pl
from jax.experimental.pallas import tpu as pltpu
from jax.experimental.pallas import tpu_sc as plsc

HID = 64
NC, NS = 2, 16
SUB = 128

_NTYPES = ["Patient", "Admission", "Diagnosis", "Medication", "Procedure", "LabTest"]
_RELS = [
    ("Patient", "Admission"),
    ("Admission", "Patient"),
    ("Admission", "Diagnosis"),
    ("Diagnosis", "Admission"),
    ("Admission", "Medication"),
    ("Medication", "Admission"),
    ("Admission", "Procedure"),
    ("Procedure", "Admission"),
    ("Admission", "LabTest"),
    ("LabTest", "Admission"),
]
_PADN = {"Patient": 10240, "Admission": 50176, "Diagnosis": 2048,
         "Medication": 1024, "Procedure": 2048, "LabTest": 1024}
HADM = _PADN["Admission"] // 2
TRASH_F = 256
TRASH_R = 8

_INV_BN = 1.0 / math.sqrt(1.0 + 1e-5)

_FWD = [(s, d) for (s, d) in _RELS if d == "Admission"]
_REV = [(s, d) for (s, d) in _RELS if s == "Admission"]


def _rk(s, d):
    return s + "__" + d


def _cdiv(a, b):
    return -(-a // b)


def _fill_zeros(ref, rows, width):
    z = jnp.zeros((16,), jnp.float32)
    for i in range(rows):
        for jj in range(width // 16):
            ref[i, pl.ds(jj * 16, 16)] = z


def _zero_spmem(acc, zbuf, zrows, base, share):
    off = 0
    while off < share:
        sz = min(zrows, share - off)
        src = zbuf if sz == zrows else zbuf.at[pl.ds(0, sz)]
        pltpu.sync_copy(src, acc.at[pl.ds(base + off, sz)])
        off += sz


def _mesh():
    return plsc.VectorSubcoreMesh(core_axis_name="c", subcore_axis_name="s",
                                  num_cores=NC, num_subcores=NS)


_SC_PARAMS = pltpu.CompilerParams(use_tc_tiling_on_sc=False)


def _adjust_dst(idx, b, cid):
    def adj(t, cy):
        v = idx[b, 1, pl.ds(t * 16, 16)]
        lv = v - cid * HADM
        ok = (lv >= 0) & (lv < HADM)
        idx[b, 1, pl.ds(t * 16, 16)] = jnp.where(
            ok, lv, HADM + (lv & (TRASH_F - 1)))
        return cy
    lax.fori_loop(0, SUB // 16, adj, 0)


@functools.lru_cache(maxsize=None)
def _make_fwd_seg(n_sub):
    nbuf = 2
    ki = _cdiv(n_sub, NS)
    kp = _cdiv(ki, nbuf)
    share = HADM // NS

    @functools.partial(
        pl.kernel, mesh=_mesh(), compiler_params=_SC_PARAMS,
        out_type=jax.ShapeDtypeStruct((2 * HADM, HID), jnp.float32),
        scratch_types=[
            pltpu.VMEM((nbuf, 2, SUB), jnp.int32),
            pltpu.VMEM((nbuf, SUB, HID), jnp.float32),
            pltpu.VMEM((32, HID), jnp.float32),
            pltpu.VMEM_SHARED((HADM + TRASH_F, HID), jnp.float32),
        ] + [pltpu.SemaphoreType.DMA] * (2 * nbuf),
    )
    def k(tab, em, out, idx, rows, zbuf, acc, *sems):
        sg = sems[:nbuf]
        ss = sems[nbuf:]
        cid = lax.axis_index("c")
        sid = lax.axis_index("s")
        _fill_zeros(zbuf, 32, HID)
        _zero_spmem(acc, zbuf, 32, sid * share, share)
        plsc.subcore_barrier()

        def pair(t, carry):
            for b in range(nbuf):
                c = sid + (nbuf * t + b) * NS

                @pl.when(c < n_sub)
                def _(b=b, c=c):
                    @pl.when(t > 0)
                    def _():
                        pltpu.make_async_copy(
                            rows.at[b], acc.at[idx.at[b, 1]], ss[b]).wait()
                    pltpu.sync_copy(em.at[c], idx.at[b])
                    pltpu.async_copy(tab.at[idx.at[b, 0]], rows.at[b], sg[b])
                    _adjust_dst(idx, b, cid)
            for b in range(nbuf):
                c = sid + (nbuf * t + b) * NS

                @pl.when(c < n_sub)
                def _(b=b):
                    pltpu.make_async_copy(
                        tab.at[idx.at[b, 0]], rows.at[b], sg[b]).wait()
                    pltpu.async_copy(rows.at[b], acc.at[idx.at[b, 1]], ss[b],
                                     add=True)
            return carry

        lax.fori_loop(0, kp, pair, 0)
        for b in range(nbuf):
            @pl.when(sid + b * NS < n_sub)
            def _(b=b):
                pltpu.make_async_copy(
                    rows.at[b], acc.at[idx.at[b, 1]], ss[b]).wait()
        plsc.subcore_barrier()
        pltpu.sync_copy(acc.at[pl.ds(sid * share, share)],
                        out.at[pl.ds(cid * HADM + sid * share, share)])

    return k


@functools.lru_cache(maxsize=None)
def _make_rev_seg(n_sub_tuple, n_acc_tuple):
    nbuf = 4
    nrel = len(n_sub_tuple)
    w = NC * NS
    scratch = [
        pltpu.VMEM((nbuf, 2, SUB), jnp.int32),
        pltpu.VMEM((nbuf, SUB, HID), jnp.float32),
        pltpu.VMEM((32, HID), jnp.float32),
    ]
    for na in n_acc_tuple:
        scratch.append(pltpu.VMEM_SHARED((na + TRASH_R, HID), jnp.float32))
    scratch.extend([pltpu.SemaphoreType.DMA] * (2 * nbuf))

    @functools.partial(
        pl.kernel, mesh=_mesh(), compiler_params=_SC_PARAMS,
        out_type=[jax.ShapeDtypeStruct((2 * na, HID), jnp.float32)
                  for na in n_acc_tuple],
        scratch_types=scratch,
    )
    def k(*refs):
        tab = refs[0]
        ems = refs[1:1 + nrel]
        outs = refs[1 + nrel:1 + 2 * nrel]
        idx, rows, zbuf = refs[1 + 2 * nrel:4 + 2 * nrel]
        accs = refs[4 + 2 * nrel:4 + 3 * nrel]
        sems = refs[4 + 3 * nrel:]
        sg = sems[:nbuf]
        ss = sems[nbuf:]

        cid = lax.axis_index("c")
        sid = lax.axis_index("s")
        wid = sid * NC + cid
        _fill_zeros(zbuf, 32, HID)
        for r in range(nrel):
            share = n_acc_tuple[r] // NS
            _zero_spmem(accs[r], zbuf, 32, sid * share, share)
        plsc.subcore_barrier()

        for r in range(nrel):
            nsb = n_sub_tuple[r]
            ki = _cdiv(nsb, w)
            kq = _cdiv(ki, nbuf)

            def quad(t, carry, r=r, nsb=nsb):
                for b in range(nbuf):
                    c = wid + (nbuf * t + b) * w

                    @pl.when(c < nsb)
                    def _(b=b, c=c, r=r):
                        @pl.when(t > 0)
                        def _():
                            pltpu.make_async_copy(
                                rows.at[b], accs[r].at[idx.at[b, 1]],
                                ss[b]).wait()
                        pltpu.sync_copy(ems[r].at[c], idx.at[b])
                        pltpu.async_copy(tab.at[idx.at[b, 0]], rows.at[b],
                                         sg[b])
                for b in range(nbuf):
                    c = wid + (nbuf * t + b) * w

                    @pl.when(c < nsb)
                    def _(b=b, r=r):
                        pltpu.make_async_copy(
                            tab.at[idx.at[b, 0]], rows.at[b], sg[b]).wait()
                        pltpu.async_copy(rows.at[b], accs[r].at[idx.at[b, 1]],
                                         ss[b], add=True)
                return carry

            lax.fori_loop(0, kq, quad, 0)
            for b in range(nbuf):
                @pl.when(wid + b * w < nsb)
                def _(b=b, r=r):
                    pltpu.make_async_copy(
                        rows.at[b], accs[r].at[idx.at[b, 1]], ss[b]).wait()
        plsc.subcore_barrier()
        for r in range(nrel):
            na = n_acc_tuple[r]
            share = na // NS
            pltpu.sync_copy(accs[r].at[pl.ds(sid * share, share)],
                            outs[r].at[pl.ds(cid * na + sid * share, share)])

    return k


@functools.lru_cache(maxsize=None)
def _make_degree(meta):
    cw = 16
    nbuf = 2
    scratch = [
        pltpu.VMEM((nbuf, 2, SUB), jnp.int32),
        pltpu.VMEM((SUB, cw), jnp.float32),
        pltpu.VMEM((256, cw), jnp.float32),
        pltpu.VMEM_SHARED((HADM + TRASH_F, cw), jnp.float32),
    ] + [pltpu.SemaphoreType.DMA] * nbuf
    out_type = [jax.ShapeDtypeStruct((2 * n_eff, cw), jnp.float32)
                for (_, n_eff, _) in meta]

    @functools.partial(pl.kernel, mesh=_mesh(), out_type=out_type,
                       compiler_params=_SC_PARAMS, scratch_types=scratch)
    def k(*refs):
        nrel = len(meta)
        ems = refs[:nrel]
        outs = refs[nrel:2 * nrel]
        idx, ones_v, zb, cnt = refs[2 * nrel:2 * nrel + 4]
        ss = refs[2 * nrel + 4:]
        cid = lax.axis_index("c")
        sid = lax.axis_index("s")
        wid = sid * NC + cid
        one = jnp.ones((16,), jnp.float32)
        for i in range(SUB):
            ones_v[i, pl.ds(0, 16)] = one
        _fill_zeros(zb, 256, cw)

        for r, (nsb, n_eff, is_fwd) in enumerate(meta):
            share = n_eff // NS
            _zero_spmem(cnt, zb, 256, sid * share, share)
            plsc.subcore_barrier()
            w = NS if is_fwd else NC * NS
            wloc = sid if is_fwd else wid
            kq = _cdiv(_cdiv(nsb, w), nbuf)

            def pair(t, carry, r=r, nsb=nsb, is_fwd=is_fwd, w=w, wloc=wloc):
                for b in range(nbuf):
                    c = wloc + (nbuf * t + b) * w

                    @pl.when(c < nsb)
                    def _(b=b, c=c):
                        @pl.when(t > 0)
                        def _():
                            pltpu.make_async_copy(
                                ones_v, cnt.at[idx.at[b, 1]], ss[b]).wait()
                        pltpu.sync_copy(ems[r].at[c], idx.at[b])
                        if is_fwd:
                            _adjust_dst(idx, b, cid)
                        pltpu.async_copy(ones_v, cnt.at[idx.at[b, 1]], ss[b],
                                         add=True)
                return carry

            lax.fori_loop(0, kq, pair, 0)
            for b in range(nbuf):
                @pl.when(wloc + b * w < nsb)
                def _(b=b):
                    pltpu.make_async_copy(
                        ones_v, cnt.at[idx.at[b, 1]], ss[b]).wait()
            plsc.subcore_barrier()
            pltpu.sync_copy(cnt.at[pl.ds(sid * share, share)],
                            outs[r].at[pl.ds(cid * n_eff + sid * share, share)])
            plsc.subcore_barrier()

    return k


_BN = 512


@functools.lru_cache(maxsize=None)
def _make_transform(n_pad, kdim):
    def body(x_ref, w_ref, b_ref, g_ref, b2_ref, o_ref):
        o = jnp.dot(x_ref[...], w_ref[...], preferred_element_type=jnp.float32)
        o = o + b_ref[...]
        o_ref[...] = o * (g_ref[...] * _INV_BN) + b2_ref[...]

    return pl.pallas_call(
        body,
        grid=(n_pad // _BN,),
        in_specs=[
            pl.BlockSpec((_BN, kdim), lambda i: (i, 0)),
            pl.BlockSpec((kdim, HID), lambda i: (0, 0)),
            pl.BlockSpec((1, HID), lambda i: (0, 0)),
            pl.BlockSpec((1, HID), lambda i: (0, 0)),
            pl.BlockSpec((1, HID), lambda i: (0, 0)),
        ],
        out_specs=pl.BlockSpec((_BN, HID), lambda i: (i, 0)),
        out_shape=jax.ShapeDtypeStruct((n_pad, HID), jnp.float32),
    )


@functools.lru_cache(maxsize=None)
def _make_rc_fwd(nrel):
    def body(*refs):
        for r in range(nrel):
            refs[nrel + r][...] = 1.0 / jnp.maximum(refs[r][...], 1.0)

    n = 2 * HADM
    return pl.pallas_call(
        body,
        grid=(n // _BN,),
        in_specs=[pl.BlockSpec((_BN, 16), lambda i: (i, 0))] * nrel,
        out_specs=[pl.BlockSpec((_BN, 16), lambda i: (i, 0))] * nrel,
        out_shape=[jax.ShapeDtypeStruct((n, 16), jnp.float32)] * nrel,
    )


@functools.lru_cache(maxsize=None)
def _make_rc_rev(n_acc_tuple):
    def body(*refs):
        nrel = len(n_acc_tuple)
        for r, na in enumerate(n_acc_tuple):
            c = refs[r][...]
            refs[nrel + r][...] = 1.0 / jnp.maximum(c[0] + c[1], 1.0)

    return pl.pallas_call(
        body,
        in_specs=[pl.BlockSpec((2, na, 16), lambda: (0, 0, 0))
                  for na in n_acc_tuple],
        out_specs=[pl.BlockSpec((na, 16), lambda: (0, 0))
                   for na in n_acc_tuple],
        out_shape=[jax.ShapeDtypeStruct((na, 16), jnp.float32)
                   for na in n_acc_tuple],
    )


@functools.lru_cache(maxsize=None)
def _make_combine_adm(nrel, final):
    n = 2 * HADM
    odim = 2 if final else HID

    def body(*refs):
        s_refs = refs[:nrel]
        rc_refs = refs[nrel:2 * nrel]
        x_ref = refs[2 * nrel]
        wl = refs[2 * nrel + 1][...]
        bl = refs[2 * nrel + 2][...]
        wr = refs[2 * nrel + 3][...]
        ep1 = refs[2 * nrel + 4]
        ep2 = refs[2 * nrel + 5]
        o_ref = refs[2 * nrel + 6]
        x = x_ref[...]
        best = None
        for r in range(nrel):
            a = s_refs[r][...] * rc_refs[r][...][:, 0:1]
            o = (jnp.dot(a, wl[r], preferred_element_type=jnp.float32)
                 + jnp.dot(x, wr[r], preferred_element_type=jnp.float32)
                 + bl[r][None, :])
            best = o if best is None else jnp.maximum(best, o)
        if final:
            h = jnp.maximum(best, 0.0)
            o_ref[...] = (jnp.dot(h, ep1[...], preferred_element_type=jnp.float32)
                          + ep2[...])
        else:
            h = best * (ep1[...] * _INV_BN) + ep2[...]
            o_ref[...] = jnp.maximum(h, 0.0)

    ep_specs = ([pl.BlockSpec((HID, 2), lambda i: (0, 0)),
                 pl.BlockSpec((1, 2), lambda i: (0, 0))] if final else
                [pl.BlockSpec((1, HID), lambda i: (0, 0)),
                 pl.BlockSpec((1, HID), lambda i: (0, 0))])
    return pl.pallas_call(
        body,
        grid=(n // _BN,),
        in_specs=(
            [pl.BlockSpec((_BN, HID), lambda i: (i, 0))] * nrel
            + [pl.BlockSpec((_BN, 16), lambda i: (i, 0))] * nrel
            + [pl.BlockSpec((_BN, HID), lambda i: (i, 0)),
               pl.BlockSpec((nrel, HID, HID), lambda i: (0, 0, 0)),
               pl.BlockSpec((nrel, HID), lambda i: (0, 0)),
               pl.BlockSpec((nrel, HID, HID), lambda i: (0, 0, 0))]
            + ep_specs),
        out_specs=pl.BlockSpec((_BN, odim), lambda i: (i, 0)),
        out_shape=jax.ShapeDtypeStruct((n, odim), jnp.float32),
    )


@functools.lru_cache(maxsize=None)
def _make_combine_small(n_acc, final):
    odim = 2 if final else HID

    def body(s_ref, rc_ref, x_ref, wl_ref, bl_ref, wr_ref, ep1, ep2, o_ref):
        s = s_ref[...]
        a = (s[0] + s[1]) * rc_ref[...][:, 0:1]
        o = (jnp.dot(a, wl_ref[...], preferred_element_type=jnp.float32)
             + jnp.dot(x_ref[...], wr_ref[...], preferred_element_type=jnp.float32)
             + bl_ref[...])
        if final:
            h = jnp.maximum(o, 0.0)
            o_ref[...] = (jnp.dot(h, ep1[...], preferred_element_type=jnp.float32)
                          + ep2[...])
        else:
            h = o * (ep1[...] * _INV_BN) + ep2[...]
            o_ref[...] = jnp.maximum(h, 0.0)

    ep_specs = ([pl.BlockSpec((HID, 2), lambda i: (0, 0)),
                 pl.BlockSpec((1, 2), lambda i: (0, 0))] if final else
                [pl.BlockSpec((1, HID), lambda i: (0, 0)),
                 pl.BlockSpec((1, HID), lambda i: (0, 0))])
    return pl.pallas_call(
        body,
        grid=(n_acc // _BN,),
        in_specs=(
            [pl.BlockSpec((2, _BN, HID), lambda i: (0, i, 0)),
             pl.BlockSpec((_BN, 16), lambda i: (i, 0)),
             pl.BlockSpec((_BN, HID), lambda i: (i, 0)),
             pl.BlockSpec((HID, HID), lambda i: (0, 0)),
             pl.BlockSpec((1, HID), lambda i: (0, 0)),
             pl.BlockSpec((HID, HID), lambda i: (0, 0))]
            + ep_specs),
        out_specs=pl.BlockSpec((_BN, odim), lambda i: (i, 0)),
        out_shape=jax.ShapeDtypeStruct((n_acc, odim), jnp.float32),
    )


def _pad_rows(x, n_pad):
    n = x.shape[0]
    if n == n_pad:
        return x
    return jnp.concatenate(
        [x, jnp.zeros((n_pad - n,) + x.shape[1:], x.dtype)], axis=0)


def _prep_edges(ei, trash_base, spread):
    e = ei.shape[1]
    c = _cdiv(e, SUB)
    ep = c * SUB
    npad = ep - e
    src = jnp.concatenate([ei[0], jnp.zeros((npad,), jnp.int32)])
    padv = trash_base + (jnp.arange(npad, dtype=jnp.int32) % spread)
    dst = jnp.concatenate([ei[1], padv])
    em = jnp.stack([src.reshape(c, SUB), dst.reshape(c, SUB)], axis=1)
    return em, c


def kernel(x_Patient, x_Admission, edges, params):
    p = params
    emats = {}
    nsubs = {}
    for (s, d) in _RELS:
        k = _rk(s, d)
        if d == "Admission":
            tb, sp = 2 * HADM, TRASH_F
        else:
            tb, sp = _PADN[d], TRASH_R
        em, c = _prep_edges(edges[k], tb, sp)
        emats[k] = em
        nsubs[k] = c

    deg_meta = tuple(
        (nsubs[_rk(s, d)], HADM if d == "Admission" else _PADN[d],
         d == "Admission")
        for (s, d) in _RELS)
    deg = _make_degree(deg_meta)(*[emats[_rk(s, d)] for (s, d) in _RELS])
    cnt = {_rk(s, d): deg[i] for i, (s, d) in enumerate(_RELS)}

    fwd_keys = [_rk(s, d) for (s, d) in _FWD]
    rev_keys = [_rk(s, d) for (s, d) in _REV]
    rc_f = _make_rc_fwd(len(fwd_keys))(*[cnt[k] for k in fwd_keys])
    rev_nacc = tuple(_PADN[d] for (_, d) in _REV)
    rc_r = _make_rc_rev(rev_nacc)(
        *[cnt[k].reshape(2, _PADN[d], 16) for k, (_, d) in zip(rev_keys, _REV)])
    rc = dict(zip(fwd_keys, rc_f))
    rc.update(zip(rev_keys, rc_r))

    tabs = {
        "Patient": _make_transform(_PADN["Patient"], 32)(
            _pad_rows(x_Patient, _PADN["Patient"]),
            p["pat_lin"]["W"], p["pat_lin"]["b"].reshape(1, HID),
            p["pat_bn"]["g"].reshape(1, HID), p["pat_bn"]["b"].reshape(1, HID)),
        "Admission": _make_transform(_PADN["Admission"], 48)(
            _pad_rows(x_Admission, _PADN["Admission"]),
            p["adm_lin"]["W"], p["adm_lin"]["b"].reshape(1, HID),
            p["adm_bn"]["g"].reshape(1, HID), p["adm_bn"]["b"].reshape(1, HID)),
    }
    for nt in ["Diagnosis", "Medication", "Procedure", "LabTest"]:
        tabs[nt] = _pad_rows(p["emb"][nt], _PADN[nt])

    rev_subs = tuple(nsubs[k] for k in rev_keys)
    out_heads = None
    for layer in ["1", "2", "3"]:
        final = layer == "3"
        pconv = p["conv"][layer]
        ssum_fwd = {}
        for (s, d) in _FWD:
            k = _rk(s, d)
            ssum_fwd[k] = _make_fwd_seg(nsubs[k])(tabs[s], emats[k])
        rev_args = [tabs["Admission"]] + [emats[k] for k in rev_keys]
        rev_outs = _make_rev_seg(rev_subs, rev_nacc)(*rev_args)
        ssum_rev = dict(zip(rev_keys, rev_outs))

        new_tabs = {}
        wl = jnp.stack([pconv[k]["Wl"] for k in fwd_keys])
        bl = jnp.stack([pconv[k]["bl"] for k in fwd_keys])
        wr = jnp.stack([pconv[k]["Wr"] for k in fwd_keys])
        if final:
            ep1 = p["lin"]["Admission"]["W"]
            ep2 = p["lin"]["Admission"]["b"].reshape(1, 2)
        else:
            ep1 = p["bn"][layer]["Admission"]["g"].reshape(1, HID)
            ep2 = p["bn"][layer]["Admission"]["b"].reshape(1, HID)
        new_tabs["Admission"] = _make_combine_adm(len(fwd_keys), final)(
            *[ssum_fwd[k] for k in fwd_keys],
            *[rc[k] for k in fwd_keys],
            tabs["Admission"], wl, bl, wr, ep1, ep2)
        for (s, d) in _REV:
            k = _rk(s, d)
            na = _PADN[d]
            if final:
                e1 = p["lin"][d]["W"]
                e2 = p["lin"][d]["b"].reshape(1, 2)
            else:
                e1 = p["bn"][layer][d]["g"].reshape(1, HID)
                e2 = p["bn"][layer][d]["b"].reshape(1, HID)
            new_tabs[d] = _make_combine_small(na, final)(
                ssum_rev[k].reshape(2, na, HID), rc[k], tabs[d],
                pconv[k]["Wl"], pconv[k]["bl"].reshape(1, HID),
                pconv[k]["Wr"], e1, e2)
        if final:
            out_heads = new_tabs
        else:
            tabs = new_tabs

    nreal = {"Patient": x_Patient.shape[0], "Admission": x_Admission.shape[0]}
    for nt in ["Diagnosis", "Medication", "Procedure", "LabTest"]:
        nreal[nt] = p["emb"][nt].shape[0]
    return tuple(out_heads[nt][:nreal[nt]] for nt in _NTYPES)


# speedup vs baseline: 6.6155x; 1.0448x over previous
import functools
import math

import jax
import jax.numpy as jnp
from jax import lax
from jax.experimental import pallas as pl
from jax.experimental.pallas import tpu as pltpu
from jax.experimental.pallas import tpu_sc as plsc

HID = 64
NC, NS = 2, 16
SUB = 128

_NTYPES = ["Patient", "Admission", "Diagnosis", "Medication", "Procedure", "LabTest"]
_RELS = [
    ("Patient", "Admission"),
    ("Admission", "Patient"),
    ("Admission", "Diagnosis"),
    ("Diagnosis", "Admission"),
    ("Admission", "Medication"),
    ("Medication", "Admission"),
    ("Admission", "Procedure"),
    ("Procedure", "Admission"),
    ("Admission", "LabTest"),
    ("LabTest", "Admission"),
]
_PADN = {"Patient": 10240, "Admission": 50176, "Diagnosis": 2048,
         "Medication": 1024, "Procedure": 2048, "LabTest": 1024}
HADM = _PADN["Admission"] // 2
TRASH_F = 256
TRASH_R = 8

_INV_BN = 1.0 / math.sqrt(1.0 + 1e-5)

_FWD = [(s, d) for (s, d) in _RELS if d == "Admission"]
_REV = [(s, d) for (s, d) in _RELS if s == "Admission"]


def _rk(s, d):
    return s + "__" + d


def _cdiv(a, b):
    return -(-a // b)


def _fill_zeros(ref, rows, width):
    z = jnp.zeros((16,), jnp.float32)
    for i in range(rows):
        for jj in range(width // 16):
            ref[i, pl.ds(jj * 16, 16)] = z


def _zero_spmem(acc, zbuf, zrows, base, share):
    off = 0
    while off < share:
        sz = min(zrows, share - off)
        src = zbuf if sz == zrows else zbuf.at[pl.ds(0, sz)]
        pltpu.sync_copy(src, acc.at[pl.ds(base + off, sz)])
        off += sz


def _mesh():
    return plsc.VectorSubcoreMesh(core_axis_name="c", subcore_axis_name="s",
                                  num_cores=NC, num_subcores=NS)


_SC_PARAMS = pltpu.CompilerParams(use_tc_tiling_on_sc=False)


def _adjust_dst(idx, b, cid):
    def adj(t, cy):
        v = idx[b, 1, pl.ds(t * 16, 16)]
        lv = v - cid * HADM
        ok = (lv >= 0) & (lv < HADM)
        idx[b, 1, pl.ds(t * 16, 16)] = jnp.where(
            ok, lv, HADM + (lv & (TRASH_F - 1)))
        return cy
    lax.fori_loop(0, SUB // 16, adj, 0)


@functools.lru_cache(maxsize=None)
def _make_fwd_seg(n_sub):
    nbuf = 3
    ki = _cdiv(n_sub, NS)
    kp = _cdiv(ki, nbuf)
    share = HADM // NS

    @functools.partial(
        pl.kernel, mesh=_mesh(), compiler_params=_SC_PARAMS,
        out_type=jax.ShapeDtypeStruct((2 * HADM, HID), jnp.float32),
        scratch_types=[
            pltpu.VMEM((nbuf, 2, SUB), jnp.int32),
            pltpu.VMEM((nbuf, SUB, HID), jnp.float32),
            pltpu.VMEM((32, HID), jnp.float32),
            pltpu.VMEM_SHARED((HADM + TRASH_F, HID), jnp.float32),
        ] + [pltpu.SemaphoreType.DMA] * (2 * nbuf),
    )
    def k(tab, em, out, idx, rows, zbuf, acc, *sems):
        sg = sems[:nbuf]
        ss = sems[nbuf:]
        cid = lax.axis_index("c")
        sid = lax.axis_index("s")
        _fill_zeros(zbuf, 32, HID)
        _zero_spmem(acc, zbuf, 32, sid * share, share)
        plsc.subcore_barrier()

        def pair(t, carry):
            for b in range(nbuf):
                c = sid + (nbuf * t + b) * NS

                @pl.when(c < n_sub)
                def _(b=b, c=c):
                    @pl.when(t > 0)
                    def _():
                        pltpu.make_async_copy(
                            rows.at[b], acc.at[idx.at[b, 1]], ss[b]).wait()
                    pltpu.sync_copy(em.at[c], idx.at[b])
                    pltpu.async_copy(tab.at[idx.at[b, 0]], rows.at[b], sg[b])
                    _adjust_dst(idx, b, cid)
            for b in range(nbuf):
                c = sid + (nbuf * t + b) * NS

                @pl.when(c < n_sub)
                def _(b=b):
                    pltpu.make_async_copy(
                        tab.at[idx.at[b, 0]], rows.at[b], sg[b]).wait()
                    pltpu.async_copy(rows.at[b], acc.at[idx.at[b, 1]], ss[b],
                                     add=True)
            return carry

        lax.fori_loop(0, kp, pair, 0)
        for b in range(nbuf):
            @pl.when(sid + b * NS < n_sub)
            def _(b=b):
                pltpu.make_async_copy(
                    rows.at[b], acc.at[idx.at[b, 1]], ss[b]).wait()
        plsc.subcore_barrier()
        pltpu.sync_copy(acc.at[pl.ds(sid * share, share)],
                        out.at[pl.ds(cid * HADM + sid * share, share)])

    return k


@functools.lru_cache(maxsize=None)
def _make_rev_seg(n_sub_tuple, n_acc_tuple):
    nbuf = 6
    nrel = len(n_sub_tuple)
    w = NC * NS
    scratch = [
        pltpu.VMEM((nbuf, 2, SUB), jnp.int32),
        pltpu.VMEM((nbuf, SUB, HID), jnp.float32),
        pltpu.VMEM((32, HID), jnp.float32),
    ]
    for na in n_acc_tuple:
        scratch.append(pltpu.VMEM_SHARED((na + TRASH_R, HID), jnp.float32))
    scratch.extend([pltpu.SemaphoreType.DMA] * (2 * nbuf))

    @functools.partial(
        pl.kernel, mesh=_mesh(), compiler_params=_SC_PARAMS,
        out_type=[jax.ShapeDtypeStruct((2 * na, HID), jnp.float32)
                  for na in n_acc_tuple],
        scratch_types=scratch,
    )
    def k(*refs):
        tab = refs[0]
        ems = refs[1:1 + nrel]
        outs = refs[1 + nrel:1 + 2 * nrel]
        idx, rows, zbuf = refs[1 + 2 * nrel:4 + 2 * nrel]
        accs = refs[4 + 2 * nrel:4 + 3 * nrel]
        sems = refs[4 + 3 * nrel:]
        sg = sems[:nbuf]
        ss = sems[nbuf:]

        cid = lax.axis_index("c")
        sid = lax.axis_index("s")
        wid = sid * NC + cid
        _fill_zeros(zbuf, 32, HID)
        for r in range(nrel):
            share = n_acc_tuple[r] // NS
            _zero_spmem(accs[r], zbuf, 32, sid * share, share)
        plsc.subcore_barrier()

        for r in range(nrel):
            nsb = n_sub_tuple[r]
            ki = _cdiv(nsb, w)
            kq = _cdiv(ki, nbuf)

            def quad(t, carry, r=r, nsb=nsb):
                for b in range(nbuf):
                    c = wid + (nbuf * t + b) * w

                    @pl.when(c < nsb)
                    def _(b=b, c=c, r=r):
                        @pl.when(t > 0)
                        def _():
                            pltpu.make_async_copy(
                                rows.at[b], accs[r].at[idx.at[b, 1]],
                                ss[b]).wait()
                        pltpu.sync_copy(ems[r].at[c], idx.at[b])
                        pltpu.async_copy(tab.at[idx.at[b, 0]], rows.at[b],
                                         sg[b])
                for b in range(nbuf):
                    c = wid + (nbuf * t + b) * w

                    @pl.when(c < nsb)
                    def _(b=b, r=r):
                        pltpu.make_async_copy(
                            tab.at[idx.at[b, 0]], rows.at[b], sg[b]).wait()
                        pltpu.async_copy(rows.at[b], accs[r].at[idx.at[b, 1]],
                                         ss[b], add=True)
                return carry

            lax.fori_loop(0, kq, quad, 0)
            for b in range(nbuf):
                @pl.when(wid + b * w < nsb)
                def _(b=b, r=r):
                    pltpu.make_async_copy(
                        rows.at[b], accs[r].at[idx.at[b, 1]], ss[b]).wait()
        plsc.subcore_barrier()
        for r in range(nrel):
            na = n_acc_tuple[r]
            share = na // NS
            pltpu.sync_copy(accs[r].at[pl.ds(sid * share, share)],
                            outs[r].at[pl.ds(cid * na + sid * share, share)])

    return k


@functools.lru_cache(maxsize=None)
def _make_degree(meta):
    cw = 16
    nbuf = 4
    scratch = [
        pltpu.VMEM((nbuf, 2, SUB), jnp.int32),
        pltpu.VMEM((SUB, cw), jnp.float32),
        pltpu.VMEM((256, cw), jnp.float32),
        pltpu.VMEM_SHARED((HADM + TRASH_F, cw), jnp.float32),
    ] + [pltpu.SemaphoreType.DMA] * nbuf
    out_type = [jax.ShapeDtypeStruct((2 * n_eff, cw), jnp.float32)
                for (_, n_eff, _) in meta]

    @functools.partial(pl.kernel, mesh=_mesh(), out_type=out_type,
                       compiler_params=_SC_PARAMS, scratch_types=scratch)
    def k(*refs):
        nrel = len(meta)
        ems = refs[:nrel]
        outs = refs[nrel:2 * nrel]
        idx, ones_v, zb, cnt = refs[2 * nrel:2 * nrel + 4]
        ss = refs[2 * nrel + 4:]
        cid = lax.axis_index("c")
        sid = lax.axis_index("s")
        wid = sid * NC + cid
        one = jnp.ones((16,), jnp.float32)
        for i in range(SUB):
            ones_v[i, pl.ds(0, 16)] = one
        _fill_zeros(zb, 256, cw)

        for r, (nsb, n_eff, is_fwd) in enumerate(meta):
            share = n_eff // NS
            _zero_spmem(cnt, zb, 256, sid * share, share)
            plsc.subcore_barrier()
            w = NS if is_fwd else NC * NS
            wloc = sid if is_fwd else wid
            kq = _cdiv(_cdiv(nsb, w), nbuf)

            def pair(t, carry, r=r, nsb=nsb, is_fwd=is_fwd, w=w, wloc=wloc):
                for b in range(nbuf):
                    c = wloc + (nbuf * t + b) * w

                    @pl.when(c < nsb)
                    def _(b=b, c=c):
                        @pl.when(t > 0)
                        def _():
                            pltpu.make_async_copy(
                                ones_v, cnt.at[idx.at[b, 1]], ss[b]).wait()
                        pltpu.sync_copy(ems[r].at[c], idx.at[b])
                        if is_fwd:
                            _adjust_dst(idx, b, cid)
                        pltpu.async_copy(ones_v, cnt.at[idx.at[b, 1]], ss[b],
                                         add=True)
                return carry

            lax.fori_loop(0, kq, pair, 0)
            for b in range(nbuf):
                @pl.when(wloc + b * w < nsb)
                def _(b=b):
                    pltpu.make_async_copy(
                        ones_v, cnt.at[idx.at[b, 1]], ss[b]).wait()
            plsc.subcore_barrier()
            pltpu.sync_copy(cnt.at[pl.ds(sid * share, share)],
                            outs[r].at[pl.ds(cid * n_eff + sid * share, share)])
            plsc.subcore_barrier()

    return k


_BN = 512


@functools.lru_cache(maxsize=None)
def _make_transform(n_pad, kdim):
    def body(x_ref, w_ref, b_ref, g_ref, b2_ref, o_ref):
        o = jnp.dot(x_ref[...], w_ref[...], preferred_element_type=jnp.float32)
        o = o + b_ref[...]
        o_ref[...] = o * (g_ref[...] * _INV_BN) + b2_ref[...]

    return pl.pallas_call(
        body,
        grid=(n_pad // _BN,),
        in_specs=[
            pl.BlockSpec((_BN, kdim), lambda i: (i, 0)),
            pl.BlockSpec((kdim, HID), lambda i: (0, 0)),
            pl.BlockSpec((1, HID), lambda i: (0, 0)),
            pl.BlockSpec((1, HID), lambda i: (0, 0)),
            pl.BlockSpec((1, HID), lambda i: (0, 0)),
        ],
        out_specs=pl.BlockSpec((_BN, HID), lambda i: (i, 0)),
        out_shape=jax.ShapeDtypeStruct((n_pad, HID), jnp.float32),
    )


@functools.lru_cache(maxsize=None)
def _make_rc_fwd(nrel):
    def body(*refs):
        for r in range(nrel):
            refs[nrel + r][...] = 1.0 / jnp.maximum(refs[r][...], 1.0)

    n = 2 * HADM
    return pl.pallas_call(
        body,
        grid=(n // _BN,),
        in_specs=[pl.BlockSpec((_BN, 16), lambda i: (i, 0))] * nrel,
        out_specs=[pl.BlockSpec((_BN, 16), lambda i: (i, 0))] * nrel,
        out_shape=[jax.ShapeDtypeStruct((n, 16), jnp.float32)] * nrel,
    )


@functools.lru_cache(maxsize=None)
def _make_rc_rev(n_acc_tuple):
    def body(*refs):
        nrel = len(n_acc_tuple)
        for r, na in enumerate(n_acc_tuple):
            c = refs[r][...]
            refs[nrel + r][...] = 1.0 / jnp.maximum(c[0] + c[1], 1.0)

    return pl.pallas_call(
        body,
        in_specs=[pl.BlockSpec((2, na, 16), lambda: (0, 0, 0))
                  for na in n_acc_tuple],
        out_specs=[pl.BlockSpec((na, 16), lambda: (0, 0))
                   for na in n_acc_tuple],
        out_shape=[jax.ShapeDtypeStruct((na, 16), jnp.float32)
                   for na in n_acc_tuple],
    )


@functools.lru_cache(maxsize=None)
def _make_combine_adm(nrel, final):
    n = 2 * HADM
    odim = 2 if final else HID

    def body(*refs):
        s_refs = refs[:nrel]
        rc_refs = refs[nrel:2 * nrel]
        x_ref = refs[2 * nrel]
        wl = refs[2 * nrel + 1][...]
        bl = refs[2 * nrel + 2][...]
        wr = refs[2 * nrel + 3][...]
        ep1 = refs[2 * nrel + 4]
        ep2 = refs[2 * nrel + 5]
        o_ref = refs[2 * nrel + 6]
        x = x_ref[...]
        best = None
        for r in range(nrel):
            a = s_refs[r][...] * rc_refs[r][...][:, 0:1]
            o = (jnp.dot(a, wl[r], preferred_element_type=jnp.float32)
                 + jnp.dot(x, wr[r], preferred_element_type=jnp.float32)
                 + bl[r][None, :])
            best = o if best is None else jnp.maximum(best, o)
        if final:
            h = jnp.maximum(best, 0.0)
            o_ref[...] = (jnp.dot(h, ep1[...], preferred_element_type=jnp.float32)
                          + ep2[...])
        else:
            h = best * (ep1[...] * _INV_BN) + ep2[...]
            o_ref[...] = jnp.maximum(h, 0.0)

    ep_specs = ([pl.BlockSpec((HID, 2), lambda i: (0, 0)),
                 pl.BlockSpec((1, 2), lambda i: (0, 0))] if final else
                [pl.BlockSpec((1, HID), lambda i: (0, 0)),
                 pl.BlockSpec((1, HID), lambda i: (0, 0))])
    return pl.pallas_call(
        body,
        grid=(n // _BN,),
        in_specs=(
            [pl.BlockSpec((_BN, HID), lambda i: (i, 0))] * nrel
            + [pl.BlockSpec((_BN, 16), lambda i: (i, 0))] * nrel
            + [pl.BlockSpec((_BN, HID), lambda i: (i, 0)),
               pl.BlockSpec((nrel, HID, HID), lambda i: (0, 0, 0)),
               pl.BlockSpec((nrel, HID), lambda i: (0, 0)),
               pl.BlockSpec((nrel, HID, HID), lambda i: (0, 0, 0))]
            + ep_specs),
        out_specs=pl.BlockSpec((_BN, odim), lambda i: (i, 0)),
        out_shape=jax.ShapeDtypeStruct((n, odim), jnp.float32),
    )


@functools.lru_cache(maxsize=None)
def _make_combine_small(n_acc, final):
    odim = 2 if final else HID

    def body(s_ref, rc_ref, x_ref, wl_ref, bl_ref, wr_ref, ep1, ep2, o_ref):
        s = s_ref[...]
        a = (s[0] + s[1]) * rc_ref[...][:, 0:1]
        o = (jnp.dot(a, wl_ref[...], preferred_element_type=jnp.float32)
             + jnp.dot(x_ref[...], wr_ref[...], preferred_element_type=jnp.float32)
             + bl_ref[...])
        if final:
            h = jnp.maximum(o, 0.0)
            o_ref[...] = (jnp.dot(h, ep1[...], preferred_element_type=jnp.float32)
                          + ep2[...])
        else:
            h = o * (ep1[...] * _INV_BN) + ep2[...]
            o_ref[...] = jnp.maximum(h, 0.0)

    ep_specs = ([pl.BlockSpec((HID, 2), lambda i: (0, 0)),
                 pl.BlockSpec((1, 2), lambda i: (0, 0))] if final else
                [pl.BlockSpec((1, HID), lambda i: (0, 0)),
                 pl.BlockSpec((1, HID), lambda i: (0, 0))])
    return pl.pallas_call(
        body,
        grid=(n_acc // _BN,),
        in_specs=(
            [pl.BlockSpec((2, _BN, HID), lambda i: (0, i, 0)),
             pl.BlockSpec((_BN, 16), lambda i: (i, 0)),
             pl.BlockSpec((_BN, HID), lambda i: (i, 0)),
             pl.BlockSpec((HID, HID), lambda i: (0, 0)),
             pl.BlockSpec((1, HID), lambda i: (0, 0)),
             pl.BlockSpec((HID, HID), lambda i: (0, 0))]
            + ep_specs),
        out_specs=pl.BlockSpec((_BN, odim), lambda i: (i, 0)),
        out_shape=jax.ShapeDtypeStruct((n_acc, odim), jnp.float32),
    )


def _pad_rows(x, n_pad):
    n = x.shape[0]
    if n == n_pad:
        return x
    return jnp.concatenate(
        [x, jnp.zeros((n_pad - n,) + x.shape[1:], x.dtype)], axis=0)


def _prep_edges(ei, trash_base, spread):
    e = ei.shape[1]
    c = _cdiv(e, SUB)
    ep = c * SUB
    npad = ep - e
    src = jnp.concatenate([ei[0], jnp.zeros((npad,), jnp.int32)])
    padv = trash_base + (jnp.arange(npad, dtype=jnp.int32) % spread)
    dst = jnp.concatenate([ei[1], padv])
    em = jnp.stack([src.reshape(c, SUB), dst.reshape(c, SUB)], axis=1)
    return em, c


def kernel(x_Patient, x_Admission, edges, params):
    p = params
    emats = {}
    nsubs = {}
    for (s, d) in _RELS:
        k = _rk(s, d)
        if d == "Admission":
            tb, sp = 2 * HADM, TRASH_F
        else:
            tb, sp = _PADN[d], TRASH_R
        em, c = _prep_edges(edges[k], tb, sp)
        emats[k] = em
        nsubs[k] = c

    deg_meta = tuple(
        (nsubs[_rk(s, d)], HADM if d == "Admission" else _PADN[d],
         d == "Admission")
        for (s, d) in _RELS)
    deg = _make_degree(deg_meta)(*[emats[_rk(s, d)] for (s, d) in _RELS])
    cnt = {_rk(s, d): deg[i] for i, (s, d) in enumerate(_RELS)}

    fwd_keys = [_rk(s, d) for (s, d) in _FWD]
    rev_keys = [_rk(s, d) for (s, d) in _REV]
    rc_f = _make_rc_fwd(len(fwd_keys))(*[cnt[k] for k in fwd_keys])
    rev_nacc = tuple(_PADN[d] for (_, d) in _REV)
    rc_r = _make_rc_rev(rev_nacc)(
        *[cnt[k].reshape(2, _PADN[d], 16) for k, (_, d) in zip(rev_keys, _REV)])
    rc = dict(zip(fwd_keys, rc_f))
    rc.update(zip(rev_keys, rc_r))

    tabs = {
        "Patient": _make_transform(_PADN["Patient"], 32)(
            _pad_rows(x_Patient, _PADN["Patient"]),
            p["pat_lin"]["W"], p["pat_lin"]["b"].reshape(1, HID),
            p["pat_bn"]["g"].reshape(1, HID), p["pat_bn"]["b"].reshape(1, HID)),
        "Admission": _make_transform(_PADN["Admission"], 48)(
            _pad_rows(x_Admission, _PADN["Admission"]),
            p["adm_lin"]["W"], p["adm_lin"]["b"].reshape(1, HID),
            p["adm_bn"]["g"].reshape(1, HID), p["adm_bn"]["b"].reshape(1, HID)),
    }
    for nt in ["Diagnosis", "Medication", "Procedure", "LabTest"]:
        tabs[nt] = _pad_rows(p["emb"][nt], _PADN[nt])

    rev_subs = tuple(nsubs[k] for k in rev_keys)
    out_heads = None
    for layer in ["1", "2", "3"]:
        final = layer == "3"
        pconv = p["conv"][layer]
        ssum_fwd = {}
        for (s, d) in _FWD:
            k = _rk(s, d)
            ssum_fwd[k] = _make_fwd_seg(nsubs[k])(tabs[s], emats[k])
        rev_args = [tabs["Admission"]] + [emats[k] for k in rev_keys]
        rev_outs = _make_rev_seg(rev_subs, rev_nacc)(*rev_args)
        ssum_rev = dict(zip(rev_keys, rev_outs))

        new_tabs = {}
        wl = jnp.stack([pconv[k]["Wl"] for k in fwd_keys])
        bl = jnp.stack([pconv[k]["bl"] for k in fwd_keys])
        wr = jnp.stack([pconv[k]["Wr"] for k in fwd_keys])
        if final:
            ep1 = p["lin"]["Admission"]["W"]
            ep2 = p["lin"]["Admission"]["b"].reshape(1, 2)
        else:
            ep1 = p["bn"][layer]["Admission"]["g"].reshape(1, HID)
            ep2 = p["bn"][layer]["Admission"]["b"].reshape(1, HID)
        new_tabs["Admission"] = _make_combine_adm(len(fwd_keys), final)(
            *[ssum_fwd[k] for k in fwd_keys],
            *[rc[k] for k in fwd_keys],
            tabs["Admission"], wl, bl, wr, ep1, ep2)
        for (s, d) in _REV:
            k = _rk(s, d)
            na = _PADN[d]
            if final:
                e1 = p["lin"][d]["W"]
                e2 = p["lin"][d]["b"].reshape(1, 2)
            else:
                e1 = p["bn"][layer][d]["g"].reshape(1, HID)
                e2 = p["bn"][layer][d]["b"].reshape(1, HID)
            new_tabs[d] = _make_combine_small(na, final)(
                ssum_rev[k].reshape(2, na, HID), rc[k], tabs[d],
                pconv[k]["Wl"], pconv[k]["bl"].reshape(1, HID),
                pconv[k]["Wr"], e1, e2)
        if final:
            out_heads = new_tabs
        else:
            tabs = new_tabs

    nreal = {"Patient": x_Patient.shape[0], "Admission": x_Admission.shape[0]}
    for nt in ["Diagnosis", "Medication", "Procedure", "LabTest"]:
        nreal[nt] = p["emb"][nt].shape[0]
    return tuple(out_heads[nt][:nreal[nt]] for nt in _NTYPES)


# speedup vs baseline: 6.9347x; 1.0483x over previous
import functools
import math

import jax
import jax.numpy as jnp
from jax import lax
from jax.experimental import pallas as pl
from jax.experimental.pallas import tpu as pltpu
from jax.experimental.pallas import tpu_sc as plsc

HID = 64
NC, NS = 2, 16
SUB = 128

_NTYPES = ["Patient", "Admission", "Diagnosis", "Medication", "Procedure", "LabTest"]
_RELS = [
    ("Patient", "Admission"),
    ("Admission", "Patient"),
    ("Admission", "Diagnosis"),
    ("Diagnosis", "Admission"),
    ("Admission", "Medication"),
    ("Medication", "Admission"),
    ("Admission", "Procedure"),
    ("Procedure", "Admission"),
    ("Admission", "LabTest"),
    ("LabTest", "Admission"),
]
_PADN = {"Patient": 10240, "Admission": 50176, "Diagnosis": 2048,
         "Medication": 1024, "Procedure": 2048, "LabTest": 1024}
HADM = _PADN["Admission"] // 2
TRASH_F = 256
TRASH_R = 8

_INV_BN = 1.0 / math.sqrt(1.0 + 1e-5)

_FWD = [(s, d) for (s, d) in _RELS if d == "Admission"]
_REV = [(s, d) for (s, d) in _RELS if s == "Admission"]


def _rk(s, d):
    return s + "__" + d


def _cdiv(a, b):
    return -(-a // b)


def _fill_zeros(ref, rows, width):
    z = jnp.zeros((16,), jnp.float32)
    for i in range(rows):
        for jj in range(width // 16):
            ref[i, pl.ds(jj * 16, 16)] = z


def _zero_spmem(acc, zbuf, zrows, base, share):
    off = 0
    while off < share:
        sz = min(zrows, share - off)
        src = zbuf if sz == zrows else zbuf.at[pl.ds(0, sz)]
        pltpu.sync_copy(src, acc.at[pl.ds(base + off, sz)])
        off += sz


def _mesh():
    return plsc.VectorSubcoreMesh(core_axis_name="c", subcore_axis_name="s",
                                  num_cores=NC, num_subcores=NS)


_SC_PARAMS = pltpu.CompilerParams(use_tc_tiling_on_sc=False)
_SC_PARAMS_NL = pltpu.CompilerParams(use_tc_tiling_on_sc=False,
                                     needs_layout_passes=False)


def _adjust_dst(idx, b, cid):
    def adj(t, cy):
        v = idx[b, 1, pl.ds(t * 16, 16)]
        lv = v - cid * HADM
        ok = (lv >= 0) & (lv < HADM)
        idx[b, 1, pl.ds(t * 16, 16)] = jnp.where(
            ok, lv, HADM + (lv & (TRASH_F - 1)))
        return cy
    lax.fori_loop(0, SUB // 16, adj, 0)


@functools.lru_cache(maxsize=None)
def _make_fwd_seg(n_sub):
    nbuf = 3
    ki = _cdiv(n_sub, NS)
    kp = _cdiv(ki, nbuf)
    share = HADM // NS

    @functools.partial(
        pl.kernel, mesh=_mesh(), compiler_params=_SC_PARAMS,
        out_type=jax.ShapeDtypeStruct((2 * HADM, HID), jnp.float32),
        scratch_types=[
            pltpu.VMEM((nbuf, 2, SUB), jnp.int32),
            pltpu.VMEM((nbuf, SUB, HID), jnp.float32),
            pltpu.VMEM((32, HID), jnp.float32),
            pltpu.VMEM_SHARED((HADM + TRASH_F, HID), jnp.float32),
        ] + [pltpu.SemaphoreType.DMA] * (2 * nbuf),
    )
    def k(tab, em, out, idx, rows, zbuf, acc, *sems):
        sg = sems[:nbuf]
        ss = sems[nbuf:]
        cid = lax.axis_index("c")
        sid = lax.axis_index("s")
        _fill_zeros(zbuf, 32, HID)
        _zero_spmem(acc, zbuf, 32, sid * share, share)
        plsc.subcore_barrier()

        def pair(t, carry):
            for b in range(nbuf):
                c = sid + (nbuf * t + b) * NS

                @pl.when(c < n_sub)
                def _(b=b, c=c):
                    @pl.when(t > 0)
                    def _():
                        pltpu.make_async_copy(
                            rows.at[b], acc.at[idx.at[b, 1]], ss[b]).wait()
                    pltpu.sync_copy(em.at[c], idx.at[b])
                    pltpu.async_copy(tab.at[idx.at[b, 0]], rows.at[b], sg[b])
                    _adjust_dst(idx, b, cid)
            for b in range(nbuf):
                c = sid + (nbuf * t + b) * NS

                @pl.when(c < n_sub)
                def _(b=b):
                    pltpu.make_async_copy(
                        tab.at[idx.at[b, 0]], rows.at[b], sg[b]).wait()
                    pltpu.async_copy(rows.at[b], acc.at[idx.at[b, 1]], ss[b],
                                     add=True)
            return carry

        lax.fori_loop(0, kp, pair, 0)
        for b in range(nbuf):
            @pl.when(sid + b * NS < n_sub)
            def _(b=b):
                pltpu.make_async_copy(
                    rows.at[b], acc.at[idx.at[b, 1]], ss[b]).wait()
        plsc.subcore_barrier()
        pltpu.sync_copy(acc.at[pl.ds(sid * share, share)],
                        out.at[pl.ds(cid * HADM + sid * share, share)])

    return k


@functools.lru_cache(maxsize=None)
def _make_rev_seg(n_sub_tuple, n_acc_tuple):
    nbuf = 6
    nrel = len(n_sub_tuple)
    w = NC * NS
    scratch = [
        pltpu.VMEM((nbuf, 2, SUB), jnp.int32),
        pltpu.VMEM((nbuf, SUB, HID), jnp.float32),
        pltpu.VMEM((32, HID), jnp.float32),
    ]
    for na in n_acc_tuple:
        scratch.append(pltpu.VMEM_SHARED((na + TRASH_R, HID), jnp.float32))
    scratch.extend([pltpu.SemaphoreType.DMA] * (2 * nbuf))

    @functools.partial(
        pl.kernel, mesh=_mesh(), compiler_params=_SC_PARAMS,
        out_type=[jax.ShapeDtypeStruct((2 * na, HID), jnp.float32)
                  for na in n_acc_tuple],
        scratch_types=scratch,
    )
    def k(*refs):
        tab = refs[0]
        ems = refs[1:1 + nrel]
        outs = refs[1 + nrel:1 + 2 * nrel]
        idx, rows, zbuf = refs[1 + 2 * nrel:4 + 2 * nrel]
        accs = refs[4 + 2 * nrel:4 + 3 * nrel]
        sems = refs[4 + 3 * nrel:]
        sg = sems[:nbuf]
        ss = sems[nbuf:]

        cid = lax.axis_index("c")
        sid = lax.axis_index("s")
        wid = sid * NC + cid
        _fill_zeros(zbuf, 32, HID)
        for r in range(nrel):
            share = n_acc_tuple[r] // NS
            _zero_spmem(accs[r], zbuf, 32, sid * share, share)
        plsc.subcore_barrier()

        for r in range(nrel):
            nsb = n_sub_tuple[r]
            ki = _cdiv(nsb, w)
            kq = _cdiv(ki, nbuf)

            def quad(t, carry, r=r, nsb=nsb):
                for b in range(nbuf):
                    c = wid + (nbuf * t + b) * w

                    @pl.when(c < nsb)
                    def _(b=b, c=c, r=r):
                        @pl.when(t > 0)
                        def _():
                            pltpu.make_async_copy(
                                rows.at[b], accs[r].at[idx.at[b, 1]],
                                ss[b]).wait()
                        pltpu.sync_copy(ems[r].at[c], idx.at[b])
                        pltpu.async_copy(tab.at[idx.at[b, 0]], rows.at[b],
                                         sg[b])
                for b in range(nbuf):
                    c = wid + (nbuf * t + b) * w

                    @pl.when(c < nsb)
                    def _(b=b, r=r):
                        pltpu.make_async_copy(
                            tab.at[idx.at[b, 0]], rows.at[b], sg[b]).wait()
                        pltpu.async_copy(rows.at[b], accs[r].at[idx.at[b, 1]],
                                         ss[b], add=True)
                return carry

            lax.fori_loop(0, kq, quad, 0)
            for b in range(nbuf):
                @pl.when(wid + b * w < nsb)
                def _(b=b, r=r):
                    pltpu.make_async_copy(
                        rows.at[b], accs[r].at[idx.at[b, 1]], ss[b]).wait()
        plsc.subcore_barrier()
        for r in range(nrel):
            na = n_acc_tuple[r]
            share = na // NS
            pltpu.sync_copy(accs[r].at[pl.ds(sid * share, share)],
                            outs[r].at[pl.ds(cid * na + sid * share, share)])

    return k


@functools.lru_cache(maxsize=None)
def _make_degree(meta):
    nbuf = 4
    w = NC * NS
    maxc = max(_cdiv(n_cnt + sp, 16) * 16 for (_, n_cnt, sp) in meta)
    scratch = [
        pltpu.VMEM((nbuf, 2, SUB), jnp.int32),
        pltpu.VMEM((maxc,), jnp.float32),
    ] + [pltpu.SemaphoreType.DMA] * nbuf
    out_type = [jax.ShapeDtypeStruct((w * n_cnt,), jnp.float32)
                for (_, n_cnt, _) in meta]

    @functools.partial(pl.kernel, mesh=_mesh(), out_type=out_type,
                       compiler_params=_SC_PARAMS_NL, scratch_types=scratch)
    def k(*refs):
        nrel = len(meta)
        ems = refs[:nrel]
        outs = refs[nrel:2 * nrel]
        idx, cnt = refs[2 * nrel:2 * nrel + 2]
        si = refs[2 * nrel + 2:]
        cid = lax.axis_index("c")
        sid = lax.axis_index("s")
        wid = sid * NC + cid
        ones16 = jnp.ones((16,), jnp.float32)
        z16 = jnp.zeros((16,), jnp.float32)

        for r, (nsb, n_cnt, sp) in enumerate(meta):
            def zero(i, cy):
                cnt[pl.ds(i * 16, 16)] = z16
                return cy
            lax.fori_loop(0, _cdiv(n_cnt + sp, 16), zero, 0)

            for b in range(nbuf):
                cb = wid + b * w

                @pl.when(cb < nsb)
                def _(b=b, cb=cb, r=r):
                    pltpu.async_copy(ems[r].at[cb], idx.at[b], si[b])

            kq = _cdiv(_cdiv(nsb, w), nbuf)

            def step(t, cy, r=r, nsb=nsb):
                for b in range(nbuf):
                    c = wid + (nbuf * t + b) * w

                    @pl.when(c < nsb)
                    def _(b=b, c=c, r=r):
                        pltpu.make_async_copy(ems[r].at[c], idx.at[b],
                                              si[b]).wait()
                        for g in range(8):
                            d16 = idx[b, 1, pl.ds(g * 16, 16)]
                            plsc.addupdate_scatter(cnt, [d16], ones16)
                        cn = c + nbuf * w

                        @pl.when(cn < nsb)
                        def _():
                            pltpu.async_copy(ems[r].at[cn], idx.at[b], si[b])
                return cy

            lax.fori_loop(0, kq, step, 0)
            pltpu.sync_copy(cnt.at[pl.ds(0, n_cnt)],
                            outs[r].at[pl.ds(wid * n_cnt, n_cnt)])

    return k


_BN = 512


@functools.lru_cache(maxsize=None)
def _make_transform(n_pad, kdim):
    def body(x_ref, w_ref, b_ref, g_ref, b2_ref, o_ref):
        o = jnp.dot(x_ref[...], w_ref[...], preferred_element_type=jnp.float32)
        o = o + b_ref[...]
        o_ref[...] = o * (g_ref[...] * _INV_BN) + b2_ref[...]

    return pl.pallas_call(
        body,
        grid=(n_pad // _BN,),
        in_specs=[
            pl.BlockSpec((_BN, kdim), lambda i: (i, 0)),
            pl.BlockSpec((kdim, HID), lambda i: (0, 0)),
            pl.BlockSpec((1, HID), lambda i: (0, 0)),
            pl.BlockSpec((1, HID), lambda i: (0, 0)),
            pl.BlockSpec((1, HID), lambda i: (0, 0)),
        ],
        out_specs=pl.BlockSpec((_BN, HID), lambda i: (i, 0)),
        out_shape=jax.ShapeDtypeStruct((n_pad, HID), jnp.float32),
    )


@functools.lru_cache(maxsize=None)
def _make_rc_fwd(nrel):
    def body(*refs):
        for r in range(nrel):
            s = jnp.sum(refs[r][...], axis=0)
            refs[nrel + r][...] = jnp.broadcast_to(
                (1.0 / jnp.maximum(s, 1.0))[:, None], (_BN, 16))

    n = 2 * HADM
    w = NC * NS
    return pl.pallas_call(
        body,
        grid=(n // _BN,),
        in_specs=[pl.BlockSpec((w, _BN), lambda i: (0, i))] * nrel,
        out_specs=[pl.BlockSpec((_BN, 16), lambda i: (i, 0))] * nrel,
        out_shape=[jax.ShapeDtypeStruct((n, 16), jnp.float32)] * nrel,
    )


@functools.lru_cache(maxsize=None)
def _make_rc_rev(n_acc_tuple):
    def body(*refs):
        nrel = len(n_acc_tuple)
        for r, na in enumerate(n_acc_tuple):
            s = jnp.sum(refs[r][...], axis=0)
            refs[nrel + r][...] = jnp.broadcast_to(
                (1.0 / jnp.maximum(s, 1.0))[:, None], (na, 16))

    w = NC * NS
    return pl.pallas_call(
        body,
        in_specs=[pl.BlockSpec((w, na), lambda: (0, 0))
                  for na in n_acc_tuple],
        out_specs=[pl.BlockSpec((na, 16), lambda: (0, 0))
                   for na in n_acc_tuple],
        out_shape=[jax.ShapeDtypeStruct((na, 16), jnp.float32)
                   for na in n_acc_tuple],
    )


@functools.lru_cache(maxsize=None)
def _make_combine_adm(nrel, final):
    n = 2 * HADM
    odim = 2 if final else HID

    def body(*refs):
        s_refs = refs[:nrel]
        rc_refs = refs[nrel:2 * nrel]
        x_ref = refs[2 * nrel]
        wl = refs[2 * nrel + 1][...]
        bl = refs[2 * nrel + 2][...]
        wr = refs[2 * nrel + 3][...]
        ep1 = refs[2 * nrel + 4]
        ep2 = refs[2 * nrel + 5]
        o_ref = refs[2 * nrel + 6]
        x = x_ref[...]
        best = None
        for r in range(nrel):
            a = s_refs[r][...] * rc_refs[r][...][:, 0:1]
            o = (jnp.dot(a, wl[r], preferred_element_type=jnp.float32)
                 + jnp.dot(x, wr[r], preferred_element_type=jnp.float32)
                 + bl[r][None, :])
            best = o if best is None else jnp.maximum(best, o)
        if final:
            h = jnp.maximum(best, 0.0)
            o_ref[...] = (jnp.dot(h, ep1[...], preferred_element_type=jnp.float32)
                          + ep2[...])
        else:
            h = best * (ep1[...] * _INV_BN) + ep2[...]
            o_ref[...] = jnp.maximum(h, 0.0)

    ep_specs = ([pl.BlockSpec((HID, 2), lambda i: (0, 0)),
                 pl.BlockSpec((1, 2), lambda i: (0, 0))] if final else
                [pl.BlockSpec((1, HID), lambda i: (0, 0)),
                 pl.BlockSpec((1, HID), lambda i: (0, 0))])
    return pl.pallas_call(
        body,
        grid=(n // _BN,),
        in_specs=(
            [pl.BlockSpec((_BN, HID), lambda i: (i, 0))] * nrel
            + [pl.BlockSpec((_BN, 16), lambda i: (i, 0))] * nrel
            + [pl.BlockSpec((_BN, HID), lambda i: (i, 0)),
               pl.BlockSpec((nrel, HID, HID), lambda i: (0, 0, 0)),
               pl.BlockSpec((nrel, HID), lambda i: (0, 0)),
               pl.BlockSpec((nrel, HID, HID), lambda i: (0, 0, 0))]
            + ep_specs),
        out_specs=pl.BlockSpec((_BN, odim), lambda i: (i, 0)),
        out_shape=jax.ShapeDtypeStruct((n, odim), jnp.float32),
    )


@functools.lru_cache(maxsize=None)
def _make_combine_small(n_acc, final):
    odim = 2 if final else HID

    def body(s_ref, rc_ref, x_ref, wl_ref, bl_ref, wr_ref, ep1, ep2, o_ref):
        s = s_ref[...]
        a = (s[0] + s[1]) * rc_ref[...][:, 0:1]
        o = (jnp.dot(a, wl_ref[...], preferred_element_type=jnp.float32)
             + jnp.dot(x_ref[...], wr_ref[...], preferred_element_type=jnp.float32)
             + bl_ref[...])
        if final:
            h = jnp.maximum(o, 0.0)
            o_ref[...] = (jnp.dot(h, ep1[...], preferred_element_type=jnp.float32)
                          + ep2[...])
        else:
            h = o * (ep1[...] * _INV_BN) + ep2[...]
            o_ref[...] = jnp.maximum(h, 0.0)

    ep_specs = ([pl.BlockSpec((HID, 2), lambda i: (0, 0)),
                 pl.BlockSpec((1, 2), lambda i: (0, 0))] if final else
                [pl.BlockSpec((1, HID), lambda i: (0, 0)),
                 pl.BlockSpec((1, HID), lambda i: (0, 0))])
    return pl.pallas_call(
        body,
        grid=(n_acc // _BN,),
        in_specs=(
            [pl.BlockSpec((2, _BN, HID), lambda i: (0, i, 0)),
             pl.BlockSpec((_BN, 16), lambda i: (i, 0)),
             pl.BlockSpec((_BN, HID), lambda i: (i, 0)),
             pl.BlockSpec((HID, HID), lambda i: (0, 0)),
             pl.BlockSpec((1, HID), lambda i: (0, 0)),
             pl.BlockSpec((HID, HID), lambda i: (0, 0))]
            + ep_specs),
        out_specs=pl.BlockSpec((_BN, odim), lambda i: (i, 0)),
        out_shape=jax.ShapeDtypeStruct((n_acc, odim), jnp.float32),
    )


def _pad_rows(x, n_pad):
    n = x.shape[0]
    if n == n_pad:
        return x
    return jnp.concatenate(
        [x, jnp.zeros((n_pad - n,) + x.shape[1:], x.dtype)], axis=0)


def _prep_edges(ei, trash_base, spread):
    e = ei.shape[1]
    c = _cdiv(e, SUB)
    ep = c * SUB
    npad = ep - e
    src = jnp.concatenate([ei[0], jnp.zeros((npad,), jnp.int32)])
    padv = trash_base + (jnp.arange(npad, dtype=jnp.int32) % spread)
    dst = jnp.concatenate([ei[1], padv])
    em = jnp.stack([src.reshape(c, SUB), dst.reshape(c, SUB)], axis=1)
    return em, c


def kernel(x_Patient, x_Admission, edges, params):
    p = params
    emats = {}
    nsubs = {}
    for (s, d) in _RELS:
        k = _rk(s, d)
        if d == "Admission":
            tb, sp = 2 * HADM, TRASH_F
        else:
            tb, sp = _PADN[d], TRASH_R
        em, c = _prep_edges(edges[k], tb, sp)
        emats[k] = em
        nsubs[k] = c

    nw = NC * NS
    deg_meta = tuple(
        (nsubs[_rk(s, d)],
         2 * HADM if d == "Admission" else _PADN[d],
         TRASH_F if d == "Admission" else TRASH_R)
        for (s, d) in _RELS)
    deg = _make_degree(deg_meta)(*[emats[_rk(s, d)] for (s, d) in _RELS])
    cnt = {_rk(s, d): deg[i].reshape(nw, deg_meta[i][1])
           for i, (s, d) in enumerate(_RELS)}

    fwd_keys = [_rk(s, d) for (s, d) in _FWD]
    rev_keys = [_rk(s, d) for (s, d) in _REV]
    rc_f = _make_rc_fwd(len(fwd_keys))(*[cnt[k] for k in fwd_keys])
    rev_nacc = tuple(_PADN[d] for (_, d) in _REV)
    rc_r = _make_rc_rev(rev_nacc)(*[cnt[k] for k in rev_keys])
    rc = dict(zip(fwd_keys, rc_f))
    rc.update(zip(rev_keys, rc_r))

    tabs = {
        "Patient": _make_transform(_PADN["Patient"], 32)(
            _pad_rows(x_Patient, _PADN["Patient"]),
            p["pat_lin"]["W"], p["pat_lin"]["b"].reshape(1, HID),
            p["pat_bn"]["g"].reshape(1, HID), p["pat_bn"]["b"].reshape(1, HID)),
        "Admission": _make_transform(_PADN["Admission"], 48)(
            _pad_rows(x_Admission, _PADN["Admission"]),
            p["adm_lin"]["W"], p["adm_lin"]["b"].reshape(1, HID),
            p["adm_bn"]["g"].reshape(1, HID), p["adm_bn"]["b"].reshape(1, HID)),
    }
    for nt in ["Diagnosis", "Medication", "Procedure", "LabTest"]:
        tabs[nt] = _pad_rows(p["emb"][nt], _PADN[nt])

    rev_subs = tuple(nsubs[k] for k in rev_keys)
    out_heads = None
    for layer in ["1", "2", "3"]:
        final = layer == "3"
        pconv = p["conv"][layer]
        ssum_fwd = {}
        for (s, d) in _FWD:
            k = _rk(s, d)
            ssum_fwd[k] = _make_fwd_seg(nsubs[k])(tabs[s], emats[k])
        rev_args = [tabs["Admission"]] + [emats[k] for k in rev_keys]
        rev_outs = _make_rev_seg(rev_subs, rev_nacc)(*rev_args)
        ssum_rev = dict(zip(rev_keys, rev_outs))

        new_tabs = {}
        wl = jnp.stack([pconv[k]["Wl"] for k in fwd_keys])
        bl = jnp.stack([pconv[k]["bl"] for k in fwd_keys])
        wr = jnp.stack([pconv[k]["Wr"] for k in fwd_keys])
        if final:
            ep1 = p["lin"]["Admission"]["W"]
            ep2 = p["lin"]["Admission"]["b"].reshape(1, 2)
        else:
            ep1 = p["bn"][layer]["Admission"]["g"].reshape(1, HID)
            ep2 = p["bn"][layer]["Admission"]["b"].reshape(1, HID)
        new_tabs["Admission"] = _make_combine_adm(len(fwd_keys), final)(
            *[ssum_fwd[k] for k in fwd_keys],
            *[rc[k] for k in fwd_keys],
            tabs["Admission"], wl, bl, wr, ep1, ep2)
        for (s, d) in _REV:
            k = _rk(s, d)
            na = _PADN[d]
            if final:
                e1 = p["lin"][d]["W"]
                e2 = p["lin"][d]["b"].reshape(1, 2)
            else:
                e1 = p["bn"][layer][d]["g"].reshape(1, HID)
                e2 = p["bn"][layer][d]["b"].reshape(1, HID)
            new_tabs[d] = _make_combine_small(na, final)(
                ssum_rev[k].reshape(2, na, HID), rc[k], tabs[d],
                pconv[k]["Wl"], pconv[k]["bl"].reshape(1, HID),
                pconv[k]["Wr"], e1, e2)
        if final:
            out_heads = new_tabs
        else:
            tabs = new_tabs

    nreal = {"Patient": x_Patient.shape[0], "Admission": x_Admission.shape[0]}
    for nt in ["Diagnosis", "Medication", "Procedure", "LabTest"]:
        nreal[nt] = p["emb"][nt].shape[0]
    return tuple(out_heads[nt][:nreal[nt]] for nt in _NTYPES)
